# baseline (speedup 1.0000x reference)
import functools
import math

import jax
import jax.numpy as jnp
from jax import lax
from jax.experimental import pallas as pl
from jax.experimental.pallas import tpu as pltpu
from jax.experimental.pallas import tpu_sc as plsc

_N = 10000
_E = 320000
_D = 128
_H = 64

_NC = 2
_NS = 16
_EPT = _E // (_NC * _NS)
_C = 128
_EPTP = 10240
_NCHUNK = _EPTP // _C
_NPAD = 10240
_RPT = _NPAD // _NS
_RZ = 32

_INV = 1.0 / math.sqrt(1.0 + 1e-5)

_ROW_BLK = 1000


_NB = 4
_NG = _NCHUNK // _NB


def _seg_sum_body(h_hbm, src_hbm, dst_hbm, out0_hbm, out1_hbm,
                  acc, srcb, dstb, rows, zbuf, gsem, ssem):
    c = lax.axis_index("c")
    s = lax.axis_index("s")
    wid = c * _NS + s

    pltpu.sync_copy(src_hbm.at[wid], srcb)
    pltpu.sync_copy(dst_hbm.at[wid], dstb)

    zv = jnp.zeros((16,), jnp.float32)

    def zrow(i, carry):
        for k in range(_H // 16):
            zbuf[i, pl.ds(16 * k, 16)] = zv
        return carry

    lax.fori_loop(0, _RZ, zrow, 0)

    def zslice(j, carry):
        pltpu.sync_copy(zbuf, acc.at[pl.ds(s * _RPT + j * _RZ, _RZ)])
        return carry

    lax.fori_loop(0, _RPT // _RZ, zslice, 0)
    plsc.subcore_barrier()

    for b in range(_NB):
        pltpu.async_copy(h_hbm.at[srcb.at[b]], rows.at[0, b], gsem)

    def grp(i, carry):
        st = lax.rem(i, 2)
        nxt = 1 - st

        @pl.when(i >= 1)
        def _():
            for b in range(_NB):
                pltpu.make_async_copy(
                    rows.at[nxt, b], acc.at[dstb.at[b]], ssem).wait()

        @pl.when(i + 1 < _NG)
        def _():
            for b in range(_NB):
                g = (i + 1) * _NB + b
                pltpu.async_copy(h_hbm.at[srcb.at[g]], rows.at[nxt, b], gsem)

        for b in range(_NB):
            pltpu.make_async_copy(
                h_hbm.at[srcb.at[b]], rows.at[st, b], gsem).wait()
        for b in range(_NB):
            g = i * _NB + b
            pltpu.async_copy(rows.at[st, b], acc.at[dstb.at[g]], ssem,
                             add=True)
        return carry

    lax.fori_loop(0, _NG, grp, 0)
    for b in range(_NB):
        pltpu.make_async_copy(
            rows.at[(_NG - 1) % 2, b], acc.at[dstb.at[b]], ssem).wait()

    plsc.subcore_barrier()

    @pl.when(c == 0)
    def _():
        pltpu.sync_copy(acc.at[pl.ds(s * _RPT, _RPT)],
                        out0_hbm.at[pl.ds(s * _RPT, _RPT)])

    @pl.when(c == 1)
    def _():
        pltpu.sync_copy(acc.at[pl.ds(s * _RPT, _RPT)],
                        out1_hbm.at[pl.ds(s * _RPT, _RPT)])


def _seg_sum(h, src3, dst3):
    mesh = plsc.VectorSubcoreMesh(core_axis_name="c", subcore_axis_name="s")
    f = functools.partial(
        pl.kernel,
        mesh=mesh,
        compiler_params=pltpu.CompilerParams(use_tc_tiling_on_sc=False),
        out_type=[jax.ShapeDtypeStruct((_NPAD, _H), jnp.float32),
                  jax.ShapeDtypeStruct((_NPAD, _H), jnp.float32)],
        scratch_types=[
            pltpu.VMEM_SHARED((_NPAD, _H), jnp.float32),
            pltpu.VMEM((_NCHUNK, _C), jnp.int32),
            pltpu.VMEM((_NCHUNK, _C), jnp.int32),
            pltpu.VMEM((2, _NB, _C, _H), jnp.float32),
            pltpu.VMEM((_RZ, _H), jnp.float32),
            pltpu.SemaphoreType.DMA,
            pltpu.SemaphoreType.DMA,
        ],
    )(_seg_sum_body)
    return f(h, src3, dst3)


def _matmul_body(x_ref, w_ref, h_ref):
    h_ref[...] = jnp.dot(x_ref[...], w_ref[...],
                         preferred_element_type=jnp.float32)


def _matmul(x, w):
    grid = _N // _ROW_BLK
    d_in = x.shape[1]
    return pl.pallas_call(
        _matmul_body,
        grid=(grid,),
        in_specs=[
            pl.BlockSpec((_ROW_BLK, d_in), lambda i: (i, 0)),
            pl.BlockSpec((d_in, _H), lambda i: (0, 0)),
        ],
        out_specs=pl.BlockSpec((_ROW_BLK, _H), lambda i: (i, 0)),
        out_shape=jax.ShapeDtypeStruct((_N, _H), jnp.float32),
    )(x, w)


def _res_body(x_ref, wr_ref, br_ref, r_ref):
    r_ref[...] = jnp.maximum(
        jnp.dot(x_ref[...], wr_ref[...], preferred_element_type=jnp.float32)
        + br_ref[...], 0.0)


def _res(x, wr, br):
    grid = _N // _ROW_BLK
    d_in = x.shape[1]
    return pl.pallas_call(
        _res_body,
        grid=(grid,),
        in_specs=[
            pl.BlockSpec((_ROW_BLK, d_in), lambda i: (i, 0)),
            pl.BlockSpec((d_in, _H), lambda i: (0, 0)),
            pl.BlockSpec((1, _H), lambda i: (0, 0)),
        ],
        out_specs=pl.BlockSpec((_ROW_BLK, _H), lambda i: (i, 0)),
        out_shape=jax.ShapeDtypeStruct((_N, _H), jnp.float32),
    )(x, wr, br.reshape(1, _H))


def _mid_body(a0_ref, a1_ref, r_ref, b_ref, g_ref, be_ref,
              w_ref, h_ref, x_ref):
    agg = a0_ref[...] + a1_ref[...]
    x = jnp.maximum(agg + b_ref[...], 0.0) + r_ref[...]
    x = g_ref[...] * (x * _INV) + be_ref[...]
    h_ref[...] = jnp.dot(x, w_ref[...], preferred_element_type=jnp.float32)
    x_ref[...] = x


def _mid(a0, a1, r, b, g, be, w):
    grid = _N // _ROW_BLK
    row = pl.BlockSpec((_ROW_BLK, _H), lambda i: (i, 0))
    vec = pl.BlockSpec((1, _H), lambda i: (0, 0))
    mat = pl.BlockSpec((_H, _H), lambda i: (0, 0))
    return pl.pallas_call(
        _mid_body,
        grid=(grid,),
        in_specs=[row, row, row, vec, vec, vec, mat],
        out_specs=[row, row],
        out_shape=[
            jax.ShapeDtypeStruct((_N, _H), jnp.float32),
            jax.ShapeDtypeStruct((_N, _H), jnp.float32),
        ],
    )(a0, a1, r, b.reshape(1, _H), g.reshape(1, _H), be.reshape(1, _H), w)


def _fin_body(a0_ref, a1_ref, r_ref, b_ref, g_ref, be_ref, o_ref):
    agg = a0_ref[...] + a1_ref[...]
    x = jnp.maximum(agg + b_ref[...], 0.0) + r_ref[...]
    o_ref[...] = g_ref[...] * (x * _INV) + be_ref[...]


def _fin(a0, a1, r, b, g, be):
    grid = _N // _ROW_BLK
    row = pl.BlockSpec((_ROW_BLK, _H), lambda i: (i, 0))
    vec = pl.BlockSpec((1, _H), lambda i: (0, 0))
    return pl.pallas_call(
        _fin_body,
        grid=(grid,),
        in_specs=[row, row, row, vec, vec, vec],
        out_specs=row,
        out_shape=jax.ShapeDtypeStruct((_N, _H), jnp.float32),
    )(a0, a1, r, b.reshape(1, _H), g.reshape(1, _H), be.reshape(1, _H))


def kernel(feats, edge_index, W1, b1, Wr1, br1, g1, be1,
           W2, b2, Wr2, br2, g2, be2):
    pad = _EPTP - _EPT
    src3 = jnp.pad(edge_index[0].reshape(_NC * _NS, _EPT),
                   ((0, 0), (0, pad))).reshape(_NC * _NS, _NCHUNK, _C)
    dst3 = jnp.pad(edge_index[1].reshape(_NC * _NS, _EPT),
                   ((0, 0), (0, pad)),
                   constant_values=_N).reshape(_NC * _NS, _NCHUNK, _C)

    h1 = _matmul(feats, W1)
    a10, a11 = _seg_sum(h1, src3, dst3)
    r1 = _res(feats, Wr1, br1)
    h2, x2 = _mid(a10, a11, r1, b1, g1, be1, W2)
    a20, a21 = _seg_sum(h2, src3, dst3)
    r2 = _res(x2, Wr2, br2)
    return _fin(a20, a21, r2, b2, g2, be2)


# speedup vs baseline: 2.5395x; 2.5395x over previous
import functools
import math

import jax
import jax.numpy as jnp
from jax import lax
from jax.experimental import pallas as pl
from jax.experimental.pallas import tpu as pltpu
from jax.experimental.pallas import tpu_sc as plsc

_N = 10000
_E = 320000
_D = 128
_H = 64

_NC = 2
_NS = 16
_EPT = _E // (_NC * _NS)
_C = 80
_NCHUNK = _EPT // _C
_NPAD = 10240
_RPT = _NPAD // _NS
_RZ = 32

_INV = 1.0 / math.sqrt(1.0 + 1e-5)

_ROW_BLK = 1000


_NB = 5
_NG = _NCHUNK // _NB


def _seg_sum_body(h_hbm, ei_hbm, out0_hbm, out1_hbm,
                  acc, srcb, dstb, rows, zbuf, gsem, ssem, dsem):
    c = lax.axis_index("c")
    s = lax.axis_index("s")
    wid = c * _NS + s
    ebase = wid * _EPT

    pltpu.sync_copy(ei_hbm.at[0, pl.ds(ebase, _EPT)], srcb)

    zv = jnp.zeros((16,), jnp.float32)

    def zrow(i, carry):
        for k in range(_H // 16):
            zbuf[i, pl.ds(16 * k, 16)] = zv
        return carry

    lax.fori_loop(0, _RZ, zrow, 0)

    def zslice(j, carry):
        pltpu.async_copy(zbuf, acc.at[pl.ds(s * _RPT + j * _RZ, _RZ)], ssem)
        return carry

    lax.fori_loop(0, _RPT // _RZ, zslice, 0)

    def zdrain(j, carry):
        pltpu.make_async_copy(zbuf, acc.at[pl.ds(s * _RPT, _RZ)], ssem).wait()
        return carry

    lax.fori_loop(0, _RPT // _RZ, zdrain, 0)
    plsc.subcore_barrier()

    def fire(i, setid):
        for b in range(_NB):
            g = i * _NB + b
            pltpu.async_copy(h_hbm.at[srcb.at[pl.ds(g * _C, _C)]],
                             rows.at[setid, b], gsem)
            pltpu.async_copy(ei_hbm.at[1, pl.ds(ebase + g * _C, _C)],
                             dstb.at[setid, b], dsem)

    fire(0, 0)

    def grp(i, carry):
        st = lax.rem(i, 2)
        nxt = 1 - st

        @pl.when(i >= 1)
        def _():
            for b in range(_NB):
                pltpu.make_async_copy(
                    rows.at[nxt, b], acc.at[dstb.at[nxt, b]], ssem).wait()

        @pl.when(i + 1 < _NG)
        def _():
            fire(i + 1, nxt)

        for b in range(_NB):
            pltpu.make_async_copy(
                h_hbm.at[srcb.at[pl.ds(b * _C, _C)]],
                rows.at[st, b], gsem).wait()
            pltpu.make_async_copy(
                ei_hbm.at[1, pl.ds(ebase, _C)],
                dstb.at[st, b], dsem).wait()
        for b in range(_NB):
            pltpu.async_copy(rows.at[st, b], acc.at[dstb.at[st, b]], ssem,
                             add=True)
        return carry

    lax.fori_loop(0, _NG, grp, 0)
    for b in range(_NB):
        pltpu.make_async_copy(
            rows.at[(_NG - 1) % 2, b],
            acc.at[dstb.at[(_NG - 1) % 2, b]], ssem).wait()

    plsc.subcore_barrier()

    @pl.when(c == 0)
    def _():
        pltpu.sync_copy(acc.at[pl.ds(s * _RPT, _RPT)],
                        out0_hbm.at[pl.ds(s * _RPT, _RPT)])

    @pl.when(c == 1)
    def _():
        pltpu.sync_copy(acc.at[pl.ds(s * _RPT, _RPT)],
                        out1_hbm.at[pl.ds(s * _RPT, _RPT)])


def _seg_sum(h, ei):
    mesh = plsc.VectorSubcoreMesh(core_axis_name="c", subcore_axis_name="s")
    f = functools.partial(
        pl.kernel,
        mesh=mesh,
        compiler_params=pltpu.CompilerParams(use_tc_tiling_on_sc=False),
        out_type=[jax.ShapeDtypeStruct((_NPAD, _H), jnp.float32),
                  jax.ShapeDtypeStruct((_NPAD, _H), jnp.float32)],
        scratch_types=[
            pltpu.VMEM_SHARED((_NPAD, _H), jnp.float32),
            pltpu.VMEM((_EPT,), jnp.int32),
            pltpu.VMEM((2, _NB, _C), jnp.int32),
            pltpu.VMEM((2, _NB, _C, _H), jnp.float32),
            pltpu.VMEM((_RZ, _H), jnp.float32),
            pltpu.SemaphoreType.DMA,
            pltpu.SemaphoreType.DMA,
            pltpu.SemaphoreType.DMA,
        ],
    )(_seg_sum_body)
    return f(h, ei)


def _matmul_body(x_ref, w_ref, h_ref):
    h_ref[...] = jnp.dot(x_ref[...], w_ref[...],
                         preferred_element_type=jnp.float32)


def _matmul(x, w):
    grid = _N // _ROW_BLK
    d_in = x.shape[1]
    return pl.pallas_call(
        _matmul_body,
        grid=(grid,),
        in_specs=[
            pl.BlockSpec((_ROW_BLK, d_in), lambda i: (i, 0)),
            pl.BlockSpec((d_in, _H), lambda i: (0, 0)),
        ],
        out_specs=pl.BlockSpec((_ROW_BLK, _H), lambda i: (i, 0)),
        out_shape=jax.ShapeDtypeStruct((_N, _H), jnp.float32),
    )(x, w)


def _res_body(x_ref, wr_ref, br_ref, r_ref):
    r_ref[...] = jnp.maximum(
        jnp.dot(x_ref[...], wr_ref[...], preferred_element_type=jnp.float32)
        + br_ref[...], 0.0)


def _res(x, wr, br):
    grid = _N // _ROW_BLK
    d_in = x.shape[1]
    return pl.pallas_call(
        _res_body,
        grid=(grid,),
        in_specs=[
            pl.BlockSpec((_ROW_BLK, d_in), lambda i: (i, 0)),
            pl.BlockSpec((d_in, _H), lambda i: (0, 0)),
            pl.BlockSpec((1, _H), lambda i: (0, 0)),
        ],
        out_specs=pl.BlockSpec((_ROW_BLK, _H), lambda i: (i, 0)),
        out_shape=jax.ShapeDtypeStruct((_N, _H), jnp.float32),
    )(x, wr, br.reshape(1, _H))


def _mid_body(a0_ref, a1_ref, r_ref, b_ref, g_ref, be_ref,
              w_ref, h_ref, x_ref):
    agg = a0_ref[...] + a1_ref[...]
    x = jnp.maximum(agg + b_ref[...], 0.0) + r_ref[...]
    x = g_ref[...] * (x * _INV) + be_ref[...]
    h_ref[...] = jnp.dot(x, w_ref[...], preferred_element_type=jnp.float32)
    x_ref[...] = x


def _mid(a0, a1, r, b, g, be, w):
    grid = _N // _ROW_BLK
    row = pl.BlockSpec((_ROW_BLK, _H), lambda i: (i, 0))
    vec = pl.BlockSpec((1, _H), lambda i: (0, 0))
    mat = pl.BlockSpec((_H, _H), lambda i: (0, 0))
    return pl.pallas_call(
        _mid_body,
        grid=(grid,),
        in_specs=[row, row, row, vec, vec, vec, mat],
        out_specs=[row, row],
        out_shape=[
            jax.ShapeDtypeStruct((_N, _H), jnp.float32),
            jax.ShapeDtypeStruct((_N, _H), jnp.float32),
        ],
    )(a0, a1, r, b.reshape(1, _H), g.reshape(1, _H), be.reshape(1, _H), w)


def _fin_body(a0_ref, a1_ref, r_ref, b_ref, g_ref, be_ref, o_ref):
    agg = a0_ref[...] + a1_ref[...]
    x = jnp.maximum(agg + b_ref[...], 0.0) + r_ref[...]
    o_ref[...] = g_ref[...] * (x * _INV) + be_ref[...]


def _fin(a0, a1, r, b, g, be):
    grid = _N // _ROW_BLK
    row = pl.BlockSpec((_ROW_BLK, _H), lambda i: (i, 0))
    vec = pl.BlockSpec((1, _H), lambda i: (0, 0))
    return pl.pallas_call(
        _fin_body,
        grid=(grid,),
        in_specs=[row, row, row, vec, vec, vec],
        out_specs=row,
        out_shape=jax.ShapeDtypeStruct((_N, _H), jnp.float32),
    )(a0, a1, r, b.reshape(1, _H), g.reshape(1, _H), be.reshape(1, _H))


def kernel(feats, edge_index, W1, b1, Wr1, br1, g1, be1,
           W2, b2, Wr2, br2, g2, be2):
    h1 = _matmul(feats, W1)
    a10, a11 = _seg_sum(h1, edge_index)
    r1 = _res(feats, Wr1, br1)
    h2, x2 = _mid(a10, a11, r1, b1, g1, be1, W2)
    a20, a21 = _seg_sum(h2, edge_index)
    r2 = _res(x2, Wr2, br2)
    return _fin(a20, a21, r2, b2, g2, be2)


# speedup vs baseline: 3.0561x; 1.2034x over previous
import functools
import math

import jax
import jax.numpy as jnp
from jax import lax
from jax.experimental import pallas as pl
from jax.experimental.pallas import tpu as pltpu
from jax.experimental.pallas import tpu_sc as plsc

_N = 10000
_E = 320000
_D = 128
_H = 64

_NC = 2
_NS = 16
_EPT = _E // (_NC * _NS)
_C = 80
_NCHUNK = _EPT // _C
_NPAD = 10240
_RPT = _NPAD // _NS
_RZ = 32

_INV = 1.0 / math.sqrt(1.0 + 1e-5)

_ROW_BLK = 2000


_NB = 5
_NG = _NCHUNK // _NB


def _seg_sum_body(h_hbm, ei_hbm, out0_hbm, out1_hbm,
                  acc, srcb, dstb, rows, zbuf, gsem, ssem, dsem):
    c = lax.axis_index("c")
    s = lax.axis_index("s")
    wid = c * _NS + s
    ebase = wid * _EPT

    pltpu.sync_copy(ei_hbm.at[0, pl.ds(ebase, _EPT)], srcb)

    zv = jnp.zeros((16,), jnp.float32)

    def zrow(i, carry):
        for k in range(_H // 16):
            zbuf[i, pl.ds(16 * k, 16)] = zv
        return carry

    lax.fori_loop(0, _RZ, zrow, 0)

    def zslice(j, carry):
        pltpu.async_copy(zbuf, acc.at[pl.ds(s * _RPT + j * _RZ, _RZ)], ssem)
        return carry

    lax.fori_loop(0, _RPT // _RZ, zslice, 0)

    def zdrain(j, carry):
        pltpu.make_async_copy(zbuf, acc.at[pl.ds(s * _RPT, _RZ)], ssem).wait()
        return carry

    lax.fori_loop(0, _RPT // _RZ, zdrain, 0)
    plsc.subcore_barrier()

    def fire(i, setid):
        for b in range(_NB):
            g = i * _NB + b
            pltpu.async_copy(h_hbm.at[srcb.at[pl.ds(g * _C, _C)]],
                             rows.at[setid, b], gsem)
            pltpu.async_copy(ei_hbm.at[1, pl.ds(ebase + g * _C, _C)],
                             dstb.at[setid, b], dsem)

    fire(0, 0)

    def grp(i, carry):
        st = lax.rem(i, 2)
        nxt = 1 - st

        @pl.when(i >= 1)
        def _():
            for b in range(_NB):
                pltpu.make_async_copy(
                    rows.at[nxt, b], acc.at[dstb.at[nxt, b]], ssem).wait()

        @pl.when(i + 1 < _NG)
        def _():
            fire(i + 1, nxt)

        for b in range(_NB):
            pltpu.make_async_copy(
                h_hbm.at[srcb.at[pl.ds(b * _C, _C)]],
                rows.at[st, b], gsem).wait()
            pltpu.make_async_copy(
                ei_hbm.at[1, pl.ds(ebase, _C)],
                dstb.at[st, b], dsem).wait()
        for b in range(_NB):
            pltpu.async_copy(rows.at[st, b], acc.at[dstb.at[st, b]], ssem,
                             add=True)
        return carry

    lax.fori_loop(0, _NG, grp, 0)
    for b in range(_NB):
        pltpu.make_async_copy(
            rows.at[(_NG - 1) % 2, b],
            acc.at[dstb.at[(_NG - 1) % 2, b]], ssem).wait()

    plsc.subcore_barrier()

    @pl.when(c == 0)
    def _():
        pltpu.sync_copy(acc.at[pl.ds(s * _RPT, _RPT)],
                        out0_hbm.at[pl.ds(s * _RPT, _RPT)])

    @pl.when(c == 1)
    def _():
        pltpu.sync_copy(acc.at[pl.ds(s * _RPT, _RPT)],
                        out1_hbm.at[pl.ds(s * _RPT, _RPT)])


def _seg_sum(h, ei):
    mesh = plsc.VectorSubcoreMesh(core_axis_name="c", subcore_axis_name="s")
    f = functools.partial(
        pl.kernel,
        mesh=mesh,
        compiler_params=pltpu.CompilerParams(use_tc_tiling_on_sc=False),
        out_type=[jax.ShapeDtypeStruct((_NPAD, _H), jnp.float32),
                  jax.ShapeDtypeStruct((_NPAD, _H), jnp.float32)],
        scratch_types=[
            pltpu.VMEM_SHARED((_NPAD, _H), jnp.float32),
            pltpu.VMEM((_EPT,), jnp.int32),
            pltpu.VMEM((2, _NB, _C), jnp.int32),
            pltpu.VMEM((2, _NB, _C, _H), jnp.float32),
            pltpu.VMEM((_RZ, _H), jnp.float32),
            pltpu.SemaphoreType.DMA,
            pltpu.SemaphoreType.DMA,
            pltpu.SemaphoreType.DMA,
        ],
    )(_seg_sum_body)
    return f(h, ei)


def _dup(v_ref):
    v = v_ref[...]
    return jnp.concatenate([v, v], axis=1)


def _pack_mm(xp, w):
    d = w.shape[0]
    lo = jnp.dot(xp[:, :d], w, preferred_element_type=jnp.float32)
    hi = jnp.dot(xp[:, d:], w, preferred_element_type=jnp.float32)
    return jnp.concatenate([lo, hi], axis=1)


def _matmul_body(x_ref, w_ref, h_ref):
    h_ref[...] = _pack_mm(x_ref[...], w_ref[...])


def _matmul(xp, w):
    grid = _N // _ROW_BLK
    d_in = w.shape[0]
    return pl.pallas_call(
        _matmul_body,
        grid=(grid,),
        in_specs=[
            pl.BlockSpec((_ROW_BLK // 2, 2 * d_in), lambda i: (i, 0)),
            pl.BlockSpec((d_in, _H), lambda i: (0, 0)),
        ],
        out_specs=pl.BlockSpec((_ROW_BLK // 2, 2 * _H), lambda i: (i, 0)),
        out_shape=jax.ShapeDtypeStruct((_N // 2, 2 * _H), jnp.float32),
    )(xp, w)


def _res_body(x_ref, wr_ref, br_ref, r_ref):
    r = _pack_mm(x_ref[...], wr_ref[...])
    r_ref[...] = jnp.maximum(r + _dup(br_ref), 0.0)


def _res(xp, wr, br):
    grid = _N // _ROW_BLK
    d_in = wr.shape[0]
    return pl.pallas_call(
        _res_body,
        grid=(grid,),
        in_specs=[
            pl.BlockSpec((_ROW_BLK // 2, 2 * d_in), lambda i: (i, 0)),
            pl.BlockSpec((d_in, _H), lambda i: (0, 0)),
            pl.BlockSpec((1, _H), lambda i: (0, 0)),
        ],
        out_specs=pl.BlockSpec((_ROW_BLK // 2, 2 * _H), lambda i: (i, 0)),
        out_shape=jax.ShapeDtypeStruct((_N // 2, 2 * _H), jnp.float32),
    )(xp, wr, br.reshape(1, _H))


def _mid_body(a0_ref, a1_ref, r_ref, b_ref, g_ref, be_ref,
              w_ref, h_ref, x_ref):
    agg = a0_ref[...] + a1_ref[...]
    x = jnp.maximum(agg + _dup(b_ref), 0.0) + r_ref[...]
    x = _dup(g_ref) * (x * _INV) + _dup(be_ref)
    h_ref[...] = _pack_mm(x, w_ref[...])
    x_ref[...] = x


def _mid(a0, a1, r, b, g, be, w):
    grid = _N // _ROW_BLK
    row = pl.BlockSpec((_ROW_BLK // 2, 2 * _H), lambda i: (i, 0))
    vec = pl.BlockSpec((1, _H), lambda i: (0, 0))
    mat = pl.BlockSpec((_H, _H), lambda i: (0, 0))
    return pl.pallas_call(
        _mid_body,
        grid=(grid,),
        in_specs=[row, row, row, vec, vec, vec, mat],
        out_specs=[row, row],
        out_shape=[
            jax.ShapeDtypeStruct((_N // 2, 2 * _H), jnp.float32),
            jax.ShapeDtypeStruct((_N // 2, 2 * _H), jnp.float32),
        ],
    )(a0, a1, r, b.reshape(1, _H), g.reshape(1, _H), be.reshape(1, _H), w)


def _fin_body(a0_ref, a1_ref, r_ref, b_ref, g_ref, be_ref, o_ref):
    agg = a0_ref[...] + a1_ref[...]
    x = jnp.maximum(agg + _dup(b_ref), 0.0) + r_ref[...]
    o_ref[...] = _dup(g_ref) * (x * _INV) + _dup(be_ref)


def _fin(a0, a1, r, b, g, be):
    grid = _N // _ROW_BLK
    row = pl.BlockSpec((_ROW_BLK // 2, 2 * _H), lambda i: (i, 0))
    vec = pl.BlockSpec((1, _H), lambda i: (0, 0))
    return pl.pallas_call(
        _fin_body,
        grid=(grid,),
        in_specs=[row, row, row, vec, vec, vec],
        out_specs=row,
        out_shape=jax.ShapeDtypeStruct((_N // 2, 2 * _H), jnp.float32),
    )(a0, a1, r, b.reshape(1, _H), g.reshape(1, _H), be.reshape(1, _H))


def kernel(feats, edge_index, W1, b1, Wr1, br1, g1, be1,
           W2, b2, Wr2, br2, g2, be2):
    featsp = feats.reshape(_N // 2, 2 * _D)
    h1p = _matmul(featsp, W1)
    a10, a11 = _seg_sum(h1p.reshape(_N, _H), edge_index)
    r1 = _res(featsp, Wr1, br1)
    pp = (_NPAD // 2, 2 * _H)
    h2p, x2p = _mid(a10.reshape(pp), a11.reshape(pp), r1, b1, g1, be1, W2)
    a20, a21 = _seg_sum(h2p.reshape(_N, _H), edge_index)
    r2 = _res(x2p, Wr2, br2)
    outp = _fin(a20.reshape(pp), a21.reshape(pp), r2, b2, g2, be2)
    return outp.reshape(_N, _H)


# speedup vs baseline: 3.1108x; 1.0179x over previous
import functools
import math

import jax
import jax.numpy as jnp
from jax import lax
from jax.experimental import pallas as pl
from jax.experimental.pallas import tpu as pltpu
from jax.experimental.pallas import tpu_sc as plsc

_N = 10000
_E = 320000
_D = 128
_H = 64

_NC = 2
_NS = 16
_EPT = _E // (_NC * _NS)
_C = 80
_NCHUNK = _EPT // _C
_NPAD = 10240
_RPT = _NPAD // _NS
_RZ = 32

_INV = 1.0 / math.sqrt(1.0 + 1e-5)

_ROW_BLK = 2000


_NB = 5
_NG = _NCHUNK // _NB


def _seg_sum_body(h_hbm, ei_hbm, out0_hbm, out1_hbm,
                  acc, srcb, dstb, rows, zbuf, gsem, ssem, dsem):
    c = lax.axis_index("c")
    s = lax.axis_index("s")
    wid = c * _NS + s
    ebase = wid * _EPT

    pltpu.sync_copy(ei_hbm.at[0, pl.ds(ebase, _EPT)], srcb)

    zv = jnp.zeros((16,), jnp.float32)

    def zrow(i, carry):
        for k in range(_H // 16):
            zbuf[i, pl.ds(16 * k, 16)] = zv
        return carry

    lax.fori_loop(0, _RZ, zrow, 0)

    def zslice(j, carry):
        pltpu.async_copy(zbuf, acc.at[pl.ds(s * _RPT + j * _RZ, _RZ)], ssem)
        return carry

    lax.fori_loop(0, _RPT // _RZ, zslice, 0)

    def zdrain(j, carry):
        pltpu.make_async_copy(zbuf, acc.at[pl.ds(s * _RPT, _RZ)], ssem).wait()
        return carry

    lax.fori_loop(0, _RPT // _RZ, zdrain, 0)
    plsc.subcore_barrier()

    def fire(i, setid):
        for b in range(_NB):
            g = i * _NB + b
            pltpu.async_copy(h_hbm.at[srcb.at[pl.ds(g * _C, _C)]],
                             rows.at[setid, b], gsem)
            pltpu.async_copy(ei_hbm.at[1, pl.ds(ebase + g * _C, _C)],
                             dstb.at[setid, b], dsem)

    fire(0, 0)

    def grp(i, carry):
        st = lax.rem(i, 2)
        nxt = 1 - st

        @pl.when(i >= 1)
        def _():
            for b in range(_NB):
                pltpu.make_async_copy(
                    rows.at[nxt, b], acc.at[dstb.at[nxt, b]], ssem).wait()

        @pl.when(i + 1 < _NG)
        def _():
            fire(i + 1, nxt)

        for b in range(_NB):
            pltpu.make_async_copy(
                h_hbm.at[srcb.at[pl.ds(b * _C, _C)]],
                rows.at[st, b], gsem).wait()
            pltpu.make_async_copy(
                ei_hbm.at[1, pl.ds(ebase, _C)],
                dstb.at[st, b], dsem).wait()
        for b in range(_NB):
            pltpu.async_copy(rows.at[st, b], acc.at[dstb.at[st, b]], ssem,
                             add=True)
        return carry

    lax.fori_loop(0, _NG, grp, 0)
    for b in range(_NB):
        pltpu.make_async_copy(
            rows.at[(_NG - 1) % 2, b],
            acc.at[dstb.at[(_NG - 1) % 2, b]], ssem).wait()

    plsc.subcore_barrier()

    @pl.when(c == 0)
    def _():
        pltpu.sync_copy(acc.at[pl.ds(s * _RPT, _RPT)],
                        out0_hbm.at[pl.ds(s * _RPT, _RPT)])

    @pl.when(c == 1)
    def _():
        pltpu.sync_copy(acc.at[pl.ds(s * _RPT, _RPT)],
                        out1_hbm.at[pl.ds(s * _RPT, _RPT)])


def _seg_sum(h, ei):
    mesh = plsc.VectorSubcoreMesh(core_axis_name="c", subcore_axis_name="s")
    f = functools.partial(
        pl.kernel,
        mesh=mesh,
        compiler_params=pltpu.CompilerParams(use_tc_tiling_on_sc=False),
        out_type=[jax.ShapeDtypeStruct((_NPAD, _H), jnp.float32),
                  jax.ShapeDtypeStruct((_NPAD, _H), jnp.float32)],
        scratch_types=[
            pltpu.VMEM_SHARED((_NPAD, _H), jnp.float32),
            pltpu.VMEM((_EPT,), jnp.int32),
            pltpu.VMEM((2, _NB, _C), jnp.int32),
            pltpu.VMEM((2, _NB, _C, _H), jnp.float32),
            pltpu.VMEM((_RZ, _H), jnp.float32),
            pltpu.SemaphoreType.DMA,
            pltpu.SemaphoreType.DMA,
            pltpu.SemaphoreType.DMA,
        ],
    )(_seg_sum_body)
    return f(h, ei)


def _dup(v_ref):
    v = v_ref[...]
    return jnp.concatenate([v, v], axis=1)


def _pack_mm(xp, w):
    d = w.shape[0]
    lo = jnp.dot(xp[:, :d], w, preferred_element_type=jnp.float32)
    hi = jnp.dot(xp[:, d:], w, preferred_element_type=jnp.float32)
    return jnp.concatenate([lo, hi], axis=1)


def _pack_mm_rows(x, w):
    v = jnp.reshape(x, (x.shape[0] // 2, 2, x.shape[1]))
    lo = jnp.dot(v[:, 0, :], w, preferred_element_type=jnp.float32)
    hi = jnp.dot(v[:, 1, :], w, preferred_element_type=jnp.float32)
    return jnp.concatenate([lo, hi], axis=1)


def _matmul_body(x_ref, w_ref, h_ref):
    h_ref[...] = _pack_mm_rows(x_ref[...], w_ref[...])


def _matmul(x, w):
    grid = _N // _ROW_BLK
    d_in = w.shape[0]
    return pl.pallas_call(
        _matmul_body,
        grid=(grid,),
        in_specs=[
            pl.BlockSpec((_ROW_BLK, d_in), lambda i: (i, 0)),
            pl.BlockSpec((d_in, _H), lambda i: (0, 0)),
        ],
        out_specs=pl.BlockSpec((_ROW_BLK // 2, 2 * _H), lambda i: (i, 0)),
        out_shape=jax.ShapeDtypeStruct((_N // 2, 2 * _H), jnp.float32),
    )(x, w)


def _res_u_body(x_ref, wr_ref, br_ref, r_ref):
    r = _pack_mm_rows(x_ref[...], wr_ref[...])
    r_ref[...] = jnp.maximum(r + _dup(br_ref), 0.0)


def _res_u(x, wr, br):
    grid = _N // _ROW_BLK
    d_in = wr.shape[0]
    return pl.pallas_call(
        _res_u_body,
        grid=(grid,),
        in_specs=[
            pl.BlockSpec((_ROW_BLK, d_in), lambda i: (i, 0)),
            pl.BlockSpec((d_in, _H), lambda i: (0, 0)),
            pl.BlockSpec((1, _H), lambda i: (0, 0)),
        ],
        out_specs=pl.BlockSpec((_ROW_BLK // 2, 2 * _H), lambda i: (i, 0)),
        out_shape=jax.ShapeDtypeStruct((_N // 2, 2 * _H), jnp.float32),
    )(x, wr, br.reshape(1, _H))


def _res_p_body(x_ref, wr_ref, br_ref, r_ref):
    r = _pack_mm(x_ref[...], wr_ref[...])
    r_ref[...] = jnp.maximum(r + _dup(br_ref), 0.0)


def _res_p(xp, wr, br):
    grid = _N // _ROW_BLK
    d_in = wr.shape[0]
    return pl.pallas_call(
        _res_p_body,
        grid=(grid,),
        in_specs=[
            pl.BlockSpec((_ROW_BLK // 2, 2 * d_in), lambda i: (i, 0)),
            pl.BlockSpec((d_in, _H), lambda i: (0, 0)),
            pl.BlockSpec((1, _H), lambda i: (0, 0)),
        ],
        out_specs=pl.BlockSpec((_ROW_BLK // 2, 2 * _H), lambda i: (i, 0)),
        out_shape=jax.ShapeDtypeStruct((_N // 2, 2 * _H), jnp.float32),
    )(xp, wr, br.reshape(1, _H))


def _mid_body(a0_ref, a1_ref, r_ref, b_ref, g_ref, be_ref,
              w_ref, h_ref, x_ref):
    agg = a0_ref[...] + a1_ref[...]
    x = jnp.maximum(agg + _dup(b_ref), 0.0) + r_ref[...]
    x = _dup(g_ref) * (x * _INV) + _dup(be_ref)
    h_ref[...] = _pack_mm(x, w_ref[...])
    x_ref[...] = x


def _mid(a0, a1, r, b, g, be, w):
    grid = _N // _ROW_BLK
    row = pl.BlockSpec((_ROW_BLK // 2, 2 * _H), lambda i: (i, 0))
    vec = pl.BlockSpec((1, _H), lambda i: (0, 0))
    mat = pl.BlockSpec((_H, _H), lambda i: (0, 0))
    return pl.pallas_call(
        _mid_body,
        grid=(grid,),
        in_specs=[row, row, row, vec, vec, vec, mat],
        out_specs=[row, row],
        out_shape=[
            jax.ShapeDtypeStruct((_N // 2, 2 * _H), jnp.float32),
            jax.ShapeDtypeStruct((_N // 2, 2 * _H), jnp.float32),
        ],
    )(a0, a1, r, b.reshape(1, _H), g.reshape(1, _H), be.reshape(1, _H), w)


def _fin_body(a0_ref, a1_ref, r_ref, b_ref, g_ref, be_ref, o_ref):
    agg = a0_ref[...] + a1_ref[...]
    x = jnp.maximum(agg + _dup(b_ref), 0.0) + r_ref[...]
    o_ref[...] = _dup(g_ref) * (x * _INV) + _dup(be_ref)


def _fin(a0, a1, r, b, g, be):
    grid = _N // _ROW_BLK
    row = pl.BlockSpec((_ROW_BLK // 2, 2 * _H), lambda i: (i, 0))
    vec = pl.BlockSpec((1, _H), lambda i: (0, 0))
    return pl.pallas_call(
        _fin_body,
        grid=(grid,),
        in_specs=[row, row, row, vec, vec, vec],
        out_specs=row,
        out_shape=jax.ShapeDtypeStruct((_N // 2, 2 * _H), jnp.float32),
    )(a0, a1, r, b.reshape(1, _H), g.reshape(1, _H), be.reshape(1, _H))


def kernel(feats, edge_index, W1, b1, Wr1, br1, g1, be1,
           W2, b2, Wr2, br2, g2, be2):
    h1p = _matmul(feats, W1)
    a10, a11 = _seg_sum(h1p.reshape(_N, _H), edge_index)
    r1 = _res_u(feats, Wr1, br1)
    pp = (_NPAD // 2, 2 * _H)
    h2p, x2p = _mid(a10.reshape(pp), a11.reshape(pp), r1, b1, g1, be1, W2)
    a20, a21 = _seg_sum(h2p.reshape(_N, _H), edge_index)
    r2 = _res_p(x2p, Wr2, br2)
    outp = _fin(a20.reshape(pp), a21.reshape(pp), r2, b2, g2, be2)
    return outp.reshape(_N, _H)


# speedup vs baseline: 3.3368x; 1.0726x over previous
import functools
import math

import jax
import jax.numpy as jnp
from jax import lax
from jax.experimental import pallas as pl
from jax.experimental.pallas import tpu as pltpu
from jax.experimental.pallas import tpu_sc as plsc

_N = 10000
_E = 320000
_D = 128
_H = 64

_NC = 2
_NS = 16
_EPT = _E // (_NC * _NS)
_C = 80
_NCHUNK = _EPT // _C
_NPAD = 10240
_RPT = _NPAD // _NS
_RZ = 32

_INV = 1.0 / math.sqrt(1.0 + 1e-5)

_ROW_BLK = 2000


_NB = 5
_NSET = 3
_NG = _NCHUNK // _NB


def _seg_sum_body(h_hbm, ei_hbm, out0_hbm, out1_hbm,
                  acc, srcb, dstb, rows, zbuf, gsem, ssem, dsem):
    c = lax.axis_index("c")
    s = lax.axis_index("s")
    wid = c * _NS + s
    ebase = wid * _EPT

    pltpu.sync_copy(ei_hbm.at[0, pl.ds(ebase, _EPT)], srcb)

    zv = jnp.zeros((16,), jnp.float32)

    def zrow(i, carry):
        for k in range(_H // 16):
            zbuf[i, pl.ds(16 * k, 16)] = zv
        return carry

    lax.fori_loop(0, _RZ, zrow, 0)

    def zslice(j, carry):
        pltpu.async_copy(zbuf, acc.at[pl.ds(s * _RPT + j * _RZ, _RZ)], ssem)
        return carry

    lax.fori_loop(0, _RPT // _RZ, zslice, 0)

    def zdrain(j, carry):
        pltpu.make_async_copy(zbuf, acc.at[pl.ds(s * _RPT, _RZ)], ssem).wait()
        return carry

    lax.fori_loop(0, _RPT // _RZ, zdrain, 0)
    plsc.subcore_barrier()

    def fire(i, setid):
        for b in range(_NB):
            g = i * _NB + b
            pltpu.async_copy(h_hbm.at[srcb.at[pl.ds(g * _C, _C)]],
                             rows.at[setid, b], gsem)
            pltpu.async_copy(ei_hbm.at[1, pl.ds(ebase + g * _C, _C)],
                             dstb.at[setid, b], dsem)

    fire(0, 0)
    fire(1, 1)

    def grp(i, carry):
        st = lax.rem(i, _NSET)
        nxt = lax.rem(i + 2, _NSET)

        @pl.when(i >= 1)
        def _():
            for b in range(_NB):
                pltpu.make_async_copy(
                    rows.at[nxt, b], acc.at[dstb.at[nxt, b]], ssem).wait()

        @pl.when(i + 2 < _NG)
        def _():
            fire(i + 2, nxt)

        for b in range(_NB):
            pltpu.make_async_copy(
                h_hbm.at[srcb.at[pl.ds(b * _C, _C)]],
                rows.at[st, b], gsem).wait()
            pltpu.make_async_copy(
                ei_hbm.at[1, pl.ds(ebase, _C)],
                dstb.at[st, b], dsem).wait()
        for b in range(_NB):
            pltpu.async_copy(rows.at[st, b], acc.at[dstb.at[st, b]], ssem,
                             add=True)
        return carry

    lax.fori_loop(0, _NG, grp, 0)
    for b in range(_NB):
        pltpu.make_async_copy(
            rows.at[(_NG - 1) % _NSET, b],
            acc.at[dstb.at[(_NG - 1) % _NSET, b]], ssem).wait()

    plsc.subcore_barrier()

    @pl.when(c == 0)
    def _():
        pltpu.sync_copy(acc.at[pl.ds(s * _RPT, _RPT)],
                        out0_hbm.at[pl.ds(s * _RPT, _RPT)])

    @pl.when(c == 1)
    def _():
        pltpu.sync_copy(acc.at[pl.ds(s * _RPT, _RPT)],
                        out1_hbm.at[pl.ds(s * _RPT, _RPT)])


def _seg_sum(h, ei):
    mesh = plsc.VectorSubcoreMesh(core_axis_name="c", subcore_axis_name="s")
    f = functools.partial(
        pl.kernel,
        mesh=mesh,
        compiler_params=pltpu.CompilerParams(use_tc_tiling_on_sc=False),
        out_type=[jax.ShapeDtypeStruct((_NPAD, _H), jnp.float32),
                  jax.ShapeDtypeStruct((_NPAD, _H), jnp.float32)],
        scratch_types=[
            pltpu.VMEM_SHARED((_NPAD, _H), jnp.float32),
            pltpu.VMEM((_EPT,), jnp.int32),
            pltpu.VMEM((_NSET, _NB, _C), jnp.int32),
            pltpu.VMEM((_NSET, _NB, _C, _H), jnp.float32),
            pltpu.VMEM((_RZ, _H), jnp.float32),
            pltpu.SemaphoreType.DMA,
            pltpu.SemaphoreType.DMA,
            pltpu.SemaphoreType.DMA,
        ],
    )(_seg_sum_body)
    return f(h, ei)


def _dup(v_ref):
    v = v_ref[...]
    return jnp.concatenate([v, v], axis=1)


def _pack_mm(xp, w):
    d = w.shape[0]
    lo = jnp.dot(xp[:, :d], w, preferred_element_type=jnp.float32)
    hi = jnp.dot(xp[:, d:], w, preferred_element_type=jnp.float32)
    return jnp.concatenate([lo, hi], axis=1)


def _pack_mm_rows(x, w):
    v = jnp.reshape(x, (x.shape[0] // 2, 2, x.shape[1]))
    lo = jnp.dot(v[:, 0, :], w, preferred_element_type=jnp.float32)
    hi = jnp.dot(v[:, 1, :], w, preferred_element_type=jnp.float32)
    return jnp.concatenate([lo, hi], axis=1)


def _matmul_body(x_ref, w_ref, h_ref):
    h_ref[...] = _pack_mm_rows(x_ref[...], w_ref[...])


def _matmul(x, w):
    grid = _N // _ROW_BLK
    d_in = w.shape[0]
    return pl.pallas_call(
        _matmul_body,
        grid=(grid,),
        in_specs=[
            pl.BlockSpec((_ROW_BLK, d_in), lambda i: (i, 0)),
            pl.BlockSpec((d_in, _H), lambda i: (0, 0)),
        ],
        out_specs=pl.BlockSpec((_ROW_BLK // 2, 2 * _H), lambda i: (i, 0)),
        out_shape=jax.ShapeDtypeStruct((_N // 2, 2 * _H), jnp.float32),
    )(x, w)


def _res_u_body(x_ref, wr_ref, br_ref, r_ref):
    r = _pack_mm_rows(x_ref[...], wr_ref[...])
    r_ref[...] = jnp.maximum(r + _dup(br_ref), 0.0)


def _res_u(x, wr, br):
    grid = _N // _ROW_BLK
    d_in = wr.shape[0]
    return pl.pallas_call(
        _res_u_body,
        grid=(grid,),
        in_specs=[
            pl.BlockSpec((_ROW_BLK, d_in), lambda i: (i, 0)),
            pl.BlockSpec((d_in, _H), lambda i: (0, 0)),
            pl.BlockSpec((1, _H), lambda i: (0, 0)),
        ],
        out_specs=pl.BlockSpec((_ROW_BLK // 2, 2 * _H), lambda i: (i, 0)),
        out_shape=jax.ShapeDtypeStruct((_N // 2, 2 * _H), jnp.float32),
    )(x, wr, br.reshape(1, _H))


def _res_p_body(x_ref, wr_ref, br_ref, r_ref):
    r = _pack_mm(x_ref[...], wr_ref[...])
    r_ref[...] = jnp.maximum(r + _dup(br_ref), 0.0)


def _res_p(xp, wr, br):
    grid = _N // _ROW_BLK
    d_in = wr.shape[0]
    return pl.pallas_call(
        _res_p_body,
        grid=(grid,),
        in_specs=[
            pl.BlockSpec((_ROW_BLK // 2, 2 * d_in), lambda i: (i, 0)),
            pl.BlockSpec((d_in, _H), lambda i: (0, 0)),
            pl.BlockSpec((1, _H), lambda i: (0, 0)),
        ],
        out_specs=pl.BlockSpec((_ROW_BLK // 2, 2 * _H), lambda i: (i, 0)),
        out_shape=jax.ShapeDtypeStruct((_N // 2, 2 * _H), jnp.float32),
    )(xp, wr, br.reshape(1, _H))


def _mid_body(a0_ref, a1_ref, r_ref, b_ref, g_ref, be_ref,
              w_ref, h_ref, x_ref):
    agg = a0_ref[...] + a1_ref[...]
    x = jnp.maximum(agg + _dup(b_ref), 0.0) + r_ref[...]
    x = _dup(g_ref) * (x * _INV) + _dup(be_ref)
    h_ref[...] = _pack_mm(x, w_ref[...])
    x_ref[...] = x


def _mid(a0, a1, r, b, g, be, w):
    grid = _N // _ROW_BLK
    row = pl.BlockSpec((_ROW_BLK // 2, 2 * _H), lambda i: (i, 0))
    vec = pl.BlockSpec((1, _H), lambda i: (0, 0))
    mat = pl.BlockSpec((_H, _H), lambda i: (0, 0))
    return pl.pallas_call(
        _mid_body,
        grid=(grid,),
        in_specs=[row, row, row, vec, vec, vec, mat],
        out_specs=[row, row],
        out_shape=[
            jax.ShapeDtypeStruct((_N // 2, 2 * _H), jnp.float32),
            jax.ShapeDtypeStruct((_N // 2, 2 * _H), jnp.float32),
        ],
    )(a0, a1, r, b.reshape(1, _H), g.reshape(1, _H), be.reshape(1, _H), w)


def _fin_body(a0_ref, a1_ref, r_ref, b_ref, g_ref, be_ref, o_ref):
    agg = a0_ref[...] + a1_ref[...]
    x = jnp.maximum(agg + _dup(b_ref), 0.0) + r_ref[...]
    o_ref[...] = _dup(g_ref) * (x * _INV) + _dup(be_ref)


def _fin(a0, a1, r, b, g, be):
    grid = _N // _ROW_BLK
    row = pl.BlockSpec((_ROW_BLK // 2, 2 * _H), lambda i: (i, 0))
    vec = pl.BlockSpec((1, _H), lambda i: (0, 0))
    return pl.pallas_call(
        _fin_body,
        grid=(grid,),
        in_specs=[row, row, row, vec, vec, vec],
        out_specs=row,
        out_shape=jax.ShapeDtypeStruct((_N // 2, 2 * _H), jnp.float32),
    )(a0, a1, r, b.reshape(1, _H), g.reshape(1, _H), be.reshape(1, _H))


def kernel(feats, edge_index, W1, b1, Wr1, br1, g1, be1,
           W2, b2, Wr2, br2, g2, be2):
    h1p = _matmul(feats, W1)
    a10, a11 = _seg_sum(h1p.reshape(_N, _H), edge_index)
    r1 = _res_u(feats, Wr1, br1)
    pp = (_NPAD // 2, 2 * _H)
    h2p, x2p = _mid(a10.reshape(pp), a11.reshape(pp), r1, b1, g1, be1, W2)
    a20, a21 = _seg_sum(h2p.reshape(_N, _H), edge_index)
    r2 = _res_p(x2p, Wr2, br2)
    outp = _fin(a20.reshape(pp), a21.reshape(pp), r2, b2, g2, be2)
    return outp.reshape(_N, _H)
